# initial kernel scaffold (unmeasured)
import jax
import jax.numpy as jnp
from jax import lax
from jax.experimental import pallas as pl
from jax.experimental.pallas import tpu as pltpu

N_DEV = 4
SQ = 1024
SKV = 1024
H_PER = 8
DH = 128
D_LOCAL = H_PER * DH
D_OUT = 1024
SCALE = 0.08838834764831843


def kernel(x, Wq, K_ext, V_ext, Wo):
    pos = lax.axis_index("i")
    wq_s = lax.dynamic_slice(Wq, (0, pos * D_LOCAL), (Wq.shape[0], D_LOCAL))
    wo_s = lax.dynamic_slice(Wo, (pos * D_LOCAL, 0), (D_LOCAL, Wo.shape[1]))
    x2 = x.reshape(SQ, x.shape[-1])
    k2 = K_ext.reshape(SKV, D_LOCAL)
    v2 = V_ext.reshape(SKV, D_LOCAL)

    def body(x_ref, wq_ref, k_ref, v_ref, wo_ref, out_ref,
             q_ref, ctx_ref, comm_ref, send_sems, recv_sems):
        my = lax.axis_index("i")
        left = lax.rem(my + N_DEV - 1, N_DEV)
        right = lax.rem(my + 1, N_DEV)

        barrier_sem = pltpu.get_barrier_semaphore()
        for nbr in (left, right):
            pl.semaphore_signal(
                barrier_sem, inc=1,
                device_id=(nbr,), device_id_type=pl.DeviceIdType.MESH,
            )
        pl.semaphore_wait(barrier_sem, 2)

        q_ref[:, :] = jnp.dot(x_ref[:, :], wq_ref[:, :],
                              preferred_element_type=jnp.float32)

        i_idx = lax.broadcasted_iota(jnp.int32, (SQ, SKV), 0)
        j_idx = lax.broadcasted_iota(jnp.int32, (SQ, SKV), 1)
        qb = i_idx // 64
        kb = j_idx // 64
        mask = (qb == kb) | (kb == 0) | (((qb + kb) % 3) == 0)

        for h in range(H_PER):
            sl = slice(h * DH, (h + 1) * DH)
            qh = q_ref[:, sl]
            kh = k_ref[:, sl]
            s = lax.dot_general(
                qh, kh, (((1,), (1,)), ((), ())),
                preferred_element_type=jnp.float32,
            ) * SCALE
            s = jnp.where(mask, s, -1e9)
            m = jnp.max(s, axis=1, keepdims=True)
            w = jnp.exp(s - m)
            w = w / jnp.sum(w, axis=1, keepdims=True)
            ctx_ref[:, sl] = jnp.dot(w, v_ref[:, sl],
                                     preferred_element_type=jnp.float32)

        partial = jnp.dot(ctx_ref[:, :], wo_ref[:, :],
                          preferred_element_type=jnp.float32)
        comm_ref[0, :, :] = partial
        out_ref[0, :, :] = partial

        for hp in range(N_DEV - 1):
            rdma = pltpu.make_async_remote_copy(
                src_ref=comm_ref.at[hp],
                dst_ref=comm_ref.at[hp + 1],
                send_sem=send_sems.at[hp],
                recv_sem=recv_sems.at[hp],
                device_id=(right,),
                device_id_type=pl.DeviceIdType.MESH,
            )
            rdma.start()
            rdma.wait()
            out_ref[0, :, :] = out_ref[0, :, :] + comm_ref[hp + 1, :, :]

    return pl.pallas_call(
        body,
        out_shape=jax.ShapeDtypeStruct((1, SQ, D_OUT), jnp.float32),
        in_specs=[pl.BlockSpec(memory_space=pltpu.VMEM)] * 5,
        out_specs=pl.BlockSpec(memory_space=pltpu.VMEM),
        scratch_shapes=[
            pltpu.VMEM((SQ, D_LOCAL), jnp.float32),
            pltpu.VMEM((SQ, D_LOCAL), jnp.float32),
            pltpu.VMEM((N_DEV, SQ, D_OUT), jnp.float32),
            pltpu.SemaphoreType.DMA((N_DEV - 1,)),
            pltpu.SemaphoreType.DMA((N_DEV - 1,)),
        ],
        compiler_params=pltpu.CompilerParams(collective_id=0),
    )(x2, wq_s, k2, v2, wo_s)


# baseline (device time: 198084 ns/iter reference)
import jax
import jax.numpy as jnp
from jax import lax
from jax.experimental import pallas as pl
from jax.experimental.pallas import tpu as pltpu

N_DEV = 4
SQ = 1024
SKV = 1024
H_PER = 8
DH = 128
D_LOCAL = H_PER * DH
D_OUT = 1024
SCALE = 0.08838834764831843


def kernel(x, Wq, K_ext, V_ext, Wo):
    pos = lax.axis_index("i")
    wq_s = lax.dynamic_slice(Wq, (0, pos * D_LOCAL), (Wq.shape[0], D_LOCAL))
    wo_s = lax.dynamic_slice(Wo, (pos * D_LOCAL, 0), (D_LOCAL, Wo.shape[1]))
    x2 = x.reshape(SQ, x.shape[-1])
    k2 = K_ext.reshape(SKV, D_LOCAL)
    v2 = V_ext.reshape(SKV, D_LOCAL)

    def body(x_ref, wq_ref, k_ref, v_ref, wo_ref, out_ref,
             q_ref, ctx_ref, comm_ref, send_sems, recv_sems):
        my = lax.axis_index("i")
        left = lax.rem(my + N_DEV - 1, N_DEV)
        right = lax.rem(my + 1, N_DEV)

        barrier_sem = pltpu.get_barrier_semaphore()
        for nbr in (left, right):
            pl.semaphore_signal(
                barrier_sem, inc=1,
                device_id=(nbr,), device_id_type=pl.DeviceIdType.MESH,
            )
        pl.semaphore_wait(barrier_sem, 2)

        q_ref[:, :] = jnp.dot(x_ref[:, :], wq_ref[:, :],
                              preferred_element_type=jnp.float32)

        i_idx = lax.broadcasted_iota(jnp.int32, (SQ, SKV), 0)
        j_idx = lax.broadcasted_iota(jnp.int32, (SQ, SKV), 1)
        qb = i_idx // 64
        kb = j_idx // 64
        mask = (qb == kb) | (kb == 0) | (((qb + kb) % 3) == 0)

        for h in range(H_PER):
            sl = slice(h * DH, (h + 1) * DH)
            qh = q_ref[:, sl]
            kh = k_ref[:, sl]
            s = lax.dot_general(
                qh, kh, (((1,), (1,)), ((), ())),
                preferred_element_type=jnp.float32,
            ) * SCALE
            s = jnp.where(mask, s, -1e9)
            m = jnp.max(s, axis=1, keepdims=True)
            w = jnp.exp(s - m)
            w = w / jnp.sum(w, axis=1, keepdims=True)
            ctx_ref[:, sl] = jnp.dot(w, v_ref[:, sl],
                                     preferred_element_type=jnp.float32)

        partial = jnp.dot(ctx_ref[:, :], wo_ref[:, :],
                          preferred_element_type=jnp.float32)
        comm_ref[0, :, :] = partial
        out_ref[0, :, :] = partial

        for hp in range(N_DEV - 1):
            rdma = pltpu.make_async_remote_copy(
                src_ref=comm_ref.at[hp],
                dst_ref=comm_ref.at[hp + 1],
                send_sem=send_sems.at[hp],
                recv_sem=recv_sems.at[hp],
                device_id=(right,),
                device_id_type=pl.DeviceIdType.MESH,
            )
            rdma.start()
            rdma.wait()
            out_ref[0, :, :] = out_ref[0, :, :] + comm_ref[hp + 1, :, :]

    return pl.pallas_call(
        body,
        out_shape=jax.ShapeDtypeStruct((1, SQ, D_OUT), jnp.float32),
        in_specs=[pl.BlockSpec(memory_space=pltpu.VMEM)] * 5,
        out_specs=pl.BlockSpec(memory_space=pltpu.VMEM),
        scratch_shapes=[
            pltpu.VMEM((SQ, D_LOCAL), jnp.float32),
            pltpu.VMEM((SQ, D_LOCAL), jnp.float32),
            pltpu.VMEM((N_DEV, SQ, D_OUT), jnp.float32),
            pltpu.SemaphoreType.DMA((N_DEV - 1,)),
            pltpu.SemaphoreType.DMA((N_DEV - 1,)),
        ],
        compiler_params=pltpu.CompilerParams(
            collective_id=0,
            vmem_limit_bytes=100 * 1024 * 1024,
        ),
    )(x2, wq_s, k2, v2, wo_s)


# device time: 100528 ns/iter; 1.9704x vs baseline; 1.9704x over previous
import jax
import jax.numpy as jnp
from jax import lax
from jax.experimental import pallas as pl
from jax.experimental.pallas import tpu as pltpu

N_DEV = 4
SQ = 1024
SKV = 1024
H_PER = 8
DH = 128
D_LOCAL = H_PER * DH
D_OUT = 1024
SCALE = 0.08838834764831843

CHUNK = SQ // N_DEV
HALF = D_OUT // 2


def kernel(x, Wq, K_ext, V_ext, Wo):
    pos = lax.axis_index("i")
    wq_s = lax.dynamic_slice(Wq, (0, pos * D_LOCAL), (Wq.shape[0], D_LOCAL))
    wo_s = lax.dynamic_slice(Wo, (pos * D_LOCAL, 0), (D_LOCAL, Wo.shape[1]))
    x2 = x.reshape(SQ, x.shape[-1])
    k2 = K_ext.reshape(SKV, D_LOCAL)
    v2 = V_ext.reshape(SKV, D_LOCAL)

    def body(x_ref, wq_ref, k_ref, v_ref, wo_ref, out_ref,
             acc_ref, recv_ref, aga_ref, agb_ref, ctx_ref,
             rs_send, rs_recv, aga_send, aga_recv, agb_send, agb_recv):
        my = lax.axis_index("i")
        left = lax.rem(my + N_DEV - 1, N_DEV)
        right = lax.rem(my + 1, N_DEV)

        barrier_sem = pltpu.get_barrier_semaphore()
        for nbr in (left, right):
            pl.semaphore_signal(
                barrier_sem, inc=1,
                device_id=(nbr,), device_id_type=pl.DeviceIdType.MESH,
            )
        pl.semaphore_wait(barrier_sem, 2)

        j_idx = lax.broadcasted_iota(jnp.int32, (CHUNK, SKV), 1)
        kb = j_idx // 64
        i_base = lax.broadcasted_iota(jnp.int32, (CHUNK, SKV), 0)

        def compute_partial(k_rel, dst_slot):
            off = (lax.rem(my - k_rel + N_DEV, N_DEV)) * CHUNK
            xc = x_ref[pl.ds(off, CHUNK), :]
            qc = jnp.dot(xc, wq_ref[:, :], preferred_element_type=jnp.float32)
            qb = (i_base + off) // 64
            mask = (qb == kb) | (kb == 0) | (((qb + kb) % 3) == 0)
            for h in range(H_PER):
                sl = slice(h * DH, (h + 1) * DH)
                s = lax.dot_general(
                    qc[:, sl], k_ref[:, sl], (((1,), (1,)), ((), ())),
                    preferred_element_type=jnp.float32,
                ) * SCALE
                s = jnp.where(mask, s, -1e9)
                m = jnp.max(s, axis=1, keepdims=True)
                w = jnp.exp(s - m)
                w = w / jnp.sum(w, axis=1, keepdims=True)
                ctx_ref[:, sl] = jnp.dot(w, v_ref[:, sl],
                                         preferred_element_type=jnp.float32)
            acc_ref[dst_slot, :, :] = jnp.dot(
                ctx_ref[:, :], wo_ref[:, :],
                preferred_element_type=jnp.float32)

        compute_partial(0, 0)
        for s in range(N_DEV - 1):
            rdma = pltpu.make_async_remote_copy(
                src_ref=acc_ref.at[s],
                dst_ref=recv_ref.at[s],
                send_sem=rs_send.at[s],
                recv_sem=rs_recv.at[s],
                device_id=(right,),
                device_id_type=pl.DeviceIdType.MESH,
            )
            rdma.start()
            compute_partial(s + 1, s + 1)
            rdma.wait()
            acc_ref[s + 1, :, :] = acc_ref[s + 1, :, :] + recv_ref[s, :, :]

        own_off = lax.rem(my + 1, N_DEV) * CHUNK
        out_ref[0, pl.ds(own_off, CHUNK), :] = acc_ref[N_DEV - 1, :, :]

        for t in range(N_DEV - 1):
            src_a = acc_ref.at[N_DEV - 1, :, pl.ds(0, HALF)] if t == 0 \
                else aga_ref.at[t - 1]
            rdma_a = pltpu.make_async_remote_copy(
                src_ref=src_a,
                dst_ref=aga_ref.at[t],
                send_sem=aga_send.at[t],
                recv_sem=aga_recv.at[t],
                device_id=(right,),
                device_id_type=pl.DeviceIdType.MESH,
            )
            src_b = acc_ref.at[N_DEV - 1, :, pl.ds(HALF, HALF)] if t == 0 \
                else agb_ref.at[t - 1]
            rdma_b = pltpu.make_async_remote_copy(
                src_ref=src_b,
                dst_ref=agb_ref.at[t],
                send_sem=agb_send.at[t],
                recv_sem=agb_recv.at[t],
                device_id=(left,),
                device_id_type=pl.DeviceIdType.MESH,
            )
            rdma_a.start()
            rdma_b.start()
            rdma_a.wait()
            rdma_b.wait()
            off_a = lax.rem(my - t + N_DEV, N_DEV) * CHUNK
            out_ref[0, pl.ds(off_a, CHUNK), 0:HALF] = aga_ref[t, :, :]
            off_b = lax.rem(my + 2 + t, N_DEV) * CHUNK
            out_ref[0, pl.ds(off_b, CHUNK), HALF:D_OUT] = agb_ref[t, :, :]

    return pl.pallas_call(
        body,
        out_shape=jax.ShapeDtypeStruct((1, SQ, D_OUT), jnp.float32),
        in_specs=[pl.BlockSpec(memory_space=pltpu.VMEM)] * 5,
        out_specs=pl.BlockSpec(memory_space=pltpu.VMEM),
        scratch_shapes=[
            pltpu.VMEM((N_DEV, CHUNK, D_OUT), jnp.float32),
            pltpu.VMEM((N_DEV - 1, CHUNK, D_OUT), jnp.float32),
            pltpu.VMEM((N_DEV - 1, CHUNK, HALF), jnp.float32),
            pltpu.VMEM((N_DEV - 1, CHUNK, HALF), jnp.float32),
            pltpu.VMEM((CHUNK, D_LOCAL), jnp.float32),
            pltpu.SemaphoreType.DMA((N_DEV - 1,)),
            pltpu.SemaphoreType.DMA((N_DEV - 1,)),
            pltpu.SemaphoreType.DMA((N_DEV - 1,)),
            pltpu.SemaphoreType.DMA((N_DEV - 1,)),
            pltpu.SemaphoreType.DMA((N_DEV - 1,)),
            pltpu.SemaphoreType.DMA((N_DEV - 1,)),
        ],
        compiler_params=pltpu.CompilerParams(
            collective_id=0,
            vmem_limit_bytes=100 * 1024 * 1024,
        ),
    )(x2, wq_s, k2, v2, wo_s)


# device time: 74669 ns/iter; 2.6528x vs baseline; 1.3463x over previous
import jax
import jax.numpy as jnp
from jax import lax
from jax.experimental import pallas as pl
from jax.experimental.pallas import tpu as pltpu

N_DEV = 4
SQ = 1024
SKV = 1024
H_PER = 8
DH = 128
D_LOCAL = H_PER * DH
D_OUT = 1024
SCALE = 0.08838834764831843

CHUNK = SQ // N_DEV
HALF = D_OUT // 2


def kernel(x, Wq, K_ext, V_ext, Wo):
    pos = lax.axis_index("i")
    wq_s = lax.dynamic_slice(Wq, (0, pos * D_LOCAL), (Wq.shape[0], D_LOCAL))
    wo_s = lax.dynamic_slice(Wo, (pos * D_LOCAL, 0), (D_LOCAL, Wo.shape[1]))
    x2 = x.reshape(SQ, x.shape[-1])
    k2 = K_ext.reshape(SKV, D_LOCAL)
    v2 = V_ext.reshape(SKV, D_LOCAL)
    bf = jnp.bfloat16
    x2, wq_s, k2, v2, wo_s = (a.astype(bf) for a in (x2, wq_s, k2, v2, wo_s))

    def body(x_ref, wq_ref, k_ref, v_ref, wo_ref, out_ref,
             acc_ref, recv_ref, aga_ref, agb_ref, ctx_ref,
             rs_send, rs_recv, aga_send, aga_recv, agb_send, agb_recv):
        my = lax.axis_index("i")
        left = lax.rem(my + N_DEV - 1, N_DEV)
        right = lax.rem(my + 1, N_DEV)

        barrier_sem = pltpu.get_barrier_semaphore()
        for nbr in (left, right):
            pl.semaphore_signal(
                barrier_sem, inc=1,
                device_id=(nbr,), device_id_type=pl.DeviceIdType.MESH,
            )
        pl.semaphore_wait(barrier_sem, 2)

        j_idx = lax.broadcasted_iota(jnp.int32, (CHUNK, SKV), 1)
        kb = j_idx // 64
        i_base = lax.broadcasted_iota(jnp.int32, (CHUNK, SKV), 0)

        def compute_partial(k_rel, dst_slot):
            off = (lax.rem(my - k_rel + N_DEV, N_DEV)) * CHUNK
            xc = x_ref[pl.ds(off, CHUNK), :]
            qc = jnp.dot(xc, wq_ref[:, :],
                         preferred_element_type=jnp.float32).astype(jnp.bfloat16)
            qb = (i_base + off) // 64
            mask = (qb == kb) | (kb == 0) | (((qb + kb) % 3) == 0)
            for h in range(H_PER):
                sl = slice(h * DH, (h + 1) * DH)
                s = lax.dot_general(
                    qc[:, sl], k_ref[:, sl], (((1,), (1,)), ((), ())),
                    preferred_element_type=jnp.float32,
                ) * SCALE
                s = jnp.where(mask, s, -1e9)
                m = jnp.max(s, axis=1, keepdims=True)
                w = jnp.exp(s - m)
                w = (w / jnp.sum(w, axis=1, keepdims=True)).astype(jnp.bfloat16)
                ctx_ref[:, sl] = jnp.dot(
                    w, v_ref[:, sl],
                    preferred_element_type=jnp.float32).astype(jnp.bfloat16)
            acc_ref[dst_slot, :, :] = jnp.dot(
                ctx_ref[:, :], wo_ref[:, :],
                preferred_element_type=jnp.float32).astype(jnp.bfloat16)

        compute_partial(0, 0)
        for s in range(N_DEV - 1):
            rdma = pltpu.make_async_remote_copy(
                src_ref=acc_ref.at[s],
                dst_ref=recv_ref.at[s],
                send_sem=rs_send.at[s],
                recv_sem=rs_recv.at[s],
                device_id=(right,),
                device_id_type=pl.DeviceIdType.MESH,
            )
            rdma.start()
            compute_partial(s + 1, s + 1)
            rdma.wait()
            acc_ref[s + 1, :, :] = (
                acc_ref[s + 1, :, :].astype(jnp.float32)
                + recv_ref[s, :, :].astype(jnp.float32)
            ).astype(jnp.bfloat16)

        own_off = lax.rem(my + 1, N_DEV) * CHUNK
        out_ref[0, pl.ds(own_off, CHUNK), :] = (
            acc_ref[N_DEV - 1, :, :].astype(jnp.float32))

        for t in range(N_DEV - 1):
            src_a = acc_ref.at[N_DEV - 1, :, pl.ds(0, HALF)] if t == 0 \
                else aga_ref.at[t - 1]
            rdma_a = pltpu.make_async_remote_copy(
                src_ref=src_a,
                dst_ref=aga_ref.at[t],
                send_sem=aga_send.at[t],
                recv_sem=aga_recv.at[t],
                device_id=(right,),
                device_id_type=pl.DeviceIdType.MESH,
            )
            src_b = acc_ref.at[N_DEV - 1, :, pl.ds(HALF, HALF)] if t == 0 \
                else agb_ref.at[t - 1]
            rdma_b = pltpu.make_async_remote_copy(
                src_ref=src_b,
                dst_ref=agb_ref.at[t],
                send_sem=agb_send.at[t],
                recv_sem=agb_recv.at[t],
                device_id=(left,),
                device_id_type=pl.DeviceIdType.MESH,
            )
            rdma_a.start()
            rdma_b.start()
            rdma_a.wait()
            rdma_b.wait()
            off_a = lax.rem(my - t + N_DEV, N_DEV) * CHUNK
            out_ref[0, pl.ds(off_a, CHUNK), 0:HALF] = (
                aga_ref[t, :, :].astype(jnp.float32))
            off_b = lax.rem(my + 2 + t, N_DEV) * CHUNK
            out_ref[0, pl.ds(off_b, CHUNK), HALF:D_OUT] = (
                agb_ref[t, :, :].astype(jnp.float32))

    return pl.pallas_call(
        body,
        out_shape=jax.ShapeDtypeStruct((1, SQ, D_OUT), jnp.float32),
        in_specs=[pl.BlockSpec(memory_space=pltpu.VMEM)] * 5,
        out_specs=pl.BlockSpec(memory_space=pltpu.VMEM),
        scratch_shapes=[
            pltpu.VMEM((N_DEV, CHUNK, D_OUT), jnp.bfloat16),
            pltpu.VMEM((N_DEV - 1, CHUNK, D_OUT), jnp.bfloat16),
            pltpu.VMEM((N_DEV - 1, CHUNK, HALF), jnp.bfloat16),
            pltpu.VMEM((N_DEV - 1, CHUNK, HALF), jnp.bfloat16),
            pltpu.VMEM((CHUNK, D_LOCAL), jnp.bfloat16),
            pltpu.SemaphoreType.DMA((N_DEV - 1,)),
            pltpu.SemaphoreType.DMA((N_DEV - 1,)),
            pltpu.SemaphoreType.DMA((N_DEV - 1,)),
            pltpu.SemaphoreType.DMA((N_DEV - 1,)),
            pltpu.SemaphoreType.DMA((N_DEV - 1,)),
            pltpu.SemaphoreType.DMA((N_DEV - 1,)),
        ],
        compiler_params=pltpu.CompilerParams(
            collective_id=0,
            vmem_limit_bytes=100 * 1024 * 1024,
        ),
    )(x2, wq_s, k2, v2, wo_s)


# device time: 71574 ns/iter; 2.7675x vs baseline; 1.0432x over previous
import jax
import jax.numpy as jnp
from jax import lax
from jax.experimental import pallas as pl
from jax.experimental.pallas import tpu as pltpu

N_DEV = 4
SQ = 1024
SKV = 1024
H_PER = 8
DH = 128
D_LOCAL = H_PER * DH
D_OUT = 1024
SCALE = 0.08838834764831843

CHUNK = SQ // N_DEV
HALF = D_OUT // 2


def kernel(x, Wq, K_ext, V_ext, Wo):
    pos = lax.axis_index("i")
    wq_s = lax.dynamic_slice(Wq, (0, pos * D_LOCAL), (Wq.shape[0], D_LOCAL))
    wo_s = lax.dynamic_slice(Wo, (pos * D_LOCAL, 0), (D_LOCAL, Wo.shape[1]))
    x2 = x.reshape(SQ, x.shape[-1])
    k2 = K_ext.reshape(SKV, D_LOCAL)
    v2 = V_ext.reshape(SKV, D_LOCAL)
    wq_s = wq_s.astype(jnp.bfloat16)
    wo_s = wo_s.astype(jnp.bfloat16)

    def body(x_ref, wq_ref, k_ref, v_ref, wo_ref, out_ref,
             acc_ref, recv_ref, aga_ref, agb_ref, ctx_ref, kb_ref, vb_ref,
             rs_send, rs_recv, aga_send, aga_recv, agb_send, agb_recv):
        my = lax.axis_index("i")
        left = lax.rem(my + N_DEV - 1, N_DEV)
        right = lax.rem(my + 1, N_DEV)

        barrier_sem = pltpu.get_barrier_semaphore()
        for nbr in (left, right):
            pl.semaphore_signal(
                barrier_sem, inc=1,
                device_id=(nbr,), device_id_type=pl.DeviceIdType.MESH,
            )
        pl.semaphore_wait(barrier_sem, 2)

        j_idx = lax.broadcasted_iota(jnp.int32, (CHUNK, SKV), 1)
        kb = j_idx // 64
        i_base = lax.broadcasted_iota(jnp.int32, (CHUNK, SKV), 0)

        kb_ref[:, :] = k_ref[:, :].astype(jnp.bfloat16)
        vb_ref[:, :] = v_ref[:, :].astype(jnp.bfloat16)

        def compute_partial(k_rel, dst_slot):
            off = (lax.rem(my - k_rel + N_DEV, N_DEV)) * CHUNK
            xc = x_ref[pl.ds(off, CHUNK), :].astype(jnp.bfloat16)
            qc = (jnp.dot(xc, wq_ref[:, :], preferred_element_type=jnp.float32)
                  * SCALE).astype(jnp.bfloat16)
            qb = (i_base + off) // 64
            mask = (qb == kb) | (kb == 0) | (((qb + kb) % 3) == 0)
            for h in range(H_PER):
                sl = slice(h * DH, (h + 1) * DH)
                s = lax.dot_general(
                    qc[:, sl], kb_ref[:, sl], (((1,), (1,)), ((), ())),
                    preferred_element_type=jnp.float32,
                )
                w = jnp.exp(jnp.where(mask, s, -1e9))
                denom = jnp.sum(w, axis=1, keepdims=True)
                ctx = jnp.dot(w.astype(jnp.bfloat16), vb_ref[:, sl],
                              preferred_element_type=jnp.float32)
                ctx_ref[:, sl] = (ctx / denom).astype(jnp.bfloat16)
            acc_ref[dst_slot, :, :] = jnp.dot(
                ctx_ref[:, :], wo_ref[:, :],
                preferred_element_type=jnp.float32).astype(jnp.bfloat16)

        compute_partial(0, 0)
        for s in range(N_DEV - 1):
            rdma = pltpu.make_async_remote_copy(
                src_ref=acc_ref.at[s],
                dst_ref=recv_ref.at[s],
                send_sem=rs_send.at[s],
                recv_sem=rs_recv.at[s],
                device_id=(right,),
                device_id_type=pl.DeviceIdType.MESH,
            )
            rdma.start()
            compute_partial(s + 1, s + 1)
            rdma.wait()
            acc_ref[s + 1, :, :] = (
                acc_ref[s + 1, :, :].astype(jnp.float32)
                + recv_ref[s, :, :].astype(jnp.float32)
            ).astype(jnp.bfloat16)

        own_off = lax.rem(my + 1, N_DEV) * CHUNK
        out_ref[0, pl.ds(own_off, CHUNK), :] = (
            acc_ref[N_DEV - 1, :, :].astype(jnp.float32))

        for t in range(N_DEV - 1):
            src_a = acc_ref.at[N_DEV - 1, :, pl.ds(0, HALF)] if t == 0 \
                else aga_ref.at[t - 1]
            rdma_a = pltpu.make_async_remote_copy(
                src_ref=src_a,
                dst_ref=aga_ref.at[t],
                send_sem=aga_send.at[t],
                recv_sem=aga_recv.at[t],
                device_id=(right,),
                device_id_type=pl.DeviceIdType.MESH,
            )
            src_b = acc_ref.at[N_DEV - 1, :, pl.ds(HALF, HALF)] if t == 0 \
                else agb_ref.at[t - 1]
            rdma_b = pltpu.make_async_remote_copy(
                src_ref=src_b,
                dst_ref=agb_ref.at[t],
                send_sem=agb_send.at[t],
                recv_sem=agb_recv.at[t],
                device_id=(left,),
                device_id_type=pl.DeviceIdType.MESH,
            )
            rdma_a.start()
            rdma_b.start()
            rdma_a.wait()
            rdma_b.wait()
            off_a = lax.rem(my - t + N_DEV, N_DEV) * CHUNK
            out_ref[0, pl.ds(off_a, CHUNK), 0:HALF] = (
                aga_ref[t, :, :].astype(jnp.float32))
            off_b = lax.rem(my + 2 + t, N_DEV) * CHUNK
            out_ref[0, pl.ds(off_b, CHUNK), HALF:D_OUT] = (
                agb_ref[t, :, :].astype(jnp.float32))

    return pl.pallas_call(
        body,
        out_shape=jax.ShapeDtypeStruct((1, SQ, D_OUT), jnp.float32),
        in_specs=[pl.BlockSpec(memory_space=pltpu.VMEM)] * 5,
        out_specs=pl.BlockSpec(memory_space=pltpu.VMEM),
        scratch_shapes=[
            pltpu.VMEM((N_DEV, CHUNK, D_OUT), jnp.bfloat16),
            pltpu.VMEM((N_DEV - 1, CHUNK, D_OUT), jnp.bfloat16),
            pltpu.VMEM((N_DEV - 1, CHUNK, HALF), jnp.bfloat16),
            pltpu.VMEM((N_DEV - 1, CHUNK, HALF), jnp.bfloat16),
            pltpu.VMEM((CHUNK, D_LOCAL), jnp.bfloat16),
            pltpu.VMEM((SKV, D_LOCAL), jnp.bfloat16),
            pltpu.VMEM((SKV, D_LOCAL), jnp.bfloat16),
            pltpu.SemaphoreType.DMA((N_DEV - 1,)),
            pltpu.SemaphoreType.DMA((N_DEV - 1,)),
            pltpu.SemaphoreType.DMA((N_DEV - 1,)),
            pltpu.SemaphoreType.DMA((N_DEV - 1,)),
            pltpu.SemaphoreType.DMA((N_DEV - 1,)),
            pltpu.SemaphoreType.DMA((N_DEV - 1,)),
        ],
        compiler_params=pltpu.CompilerParams(
            collective_id=0,
            vmem_limit_bytes=100 * 1024 * 1024,
        ),
    )(x2, wq_s, k2, v2, wo_s)


# device time: 70134 ns/iter; 2.8244x vs baseline; 1.0205x over previous
import jax
import jax.numpy as jnp
from jax import lax
from jax.experimental import pallas as pl
from jax.experimental.pallas import tpu as pltpu

N_DEV = 4
SQ = 1024
SKV = 1024
H_PER = 8
DH = 128
D_LOCAL = H_PER * DH
D_OUT = 1024
SCALE = 0.08838834764831843

CHUNK = SQ // N_DEV
HALF = D_OUT // 2


def kernel(x, Wq, K_ext, V_ext, Wo):
    pos = lax.axis_index("i")
    wq_s = lax.dynamic_slice(Wq, (0, pos * D_LOCAL), (Wq.shape[0], D_LOCAL))
    wo_s = lax.dynamic_slice(Wo, (pos * D_LOCAL, 0), (D_LOCAL, Wo.shape[1]))
    x2 = x.reshape(SQ, x.shape[-1])
    k2 = K_ext.reshape(SKV, D_LOCAL).astype(jnp.bfloat16)
    v2 = V_ext.reshape(SKV, D_LOCAL).astype(jnp.bfloat16)
    wq_s = wq_s.astype(jnp.bfloat16)
    wo_s = wo_s.astype(jnp.bfloat16)

    def body(x_ref, wq_ref, k_ref, v_ref, wo_ref, out_ref,
             acc_ref, recv_ref, aga_ref, agb_ref, ctx_ref,
             rs_send, rs_recv, aga_send, aga_recv, agb_send, agb_recv):
        my = lax.axis_index("i")
        left = lax.rem(my + N_DEV - 1, N_DEV)
        right = lax.rem(my + 1, N_DEV)

        barrier_sem = pltpu.get_barrier_semaphore()
        for nbr in (left, right):
            pl.semaphore_signal(
                barrier_sem, inc=1,
                device_id=(nbr,), device_id_type=pl.DeviceIdType.MESH,
            )
        pl.semaphore_wait(barrier_sem, 2)

        j_idx = lax.broadcasted_iota(jnp.int32, (CHUNK, SKV), 1)
        kb = j_idx // 64
        i_base = lax.broadcasted_iota(jnp.int32, (CHUNK, SKV), 0)

        def compute_partial(k_rel, dst_slot):
            off = (lax.rem(my - k_rel + N_DEV, N_DEV)) * CHUNK
            xc = x_ref[pl.ds(off, CHUNK), :].astype(jnp.bfloat16)
            qc = (jnp.dot(xc, wq_ref[:, :], preferred_element_type=jnp.float32)
                  * SCALE).astype(jnp.bfloat16)
            qb = (i_base + off) // 64
            mask = (qb == kb) | (kb == 0) | (((qb + kb) % 3) == 0)
            for h in range(H_PER):
                sl = slice(h * DH, (h + 1) * DH)
                s = lax.dot_general(
                    qc[:, sl], k_ref[:, sl], (((1,), (1,)), ((), ())),
                    preferred_element_type=jnp.float32,
                )
                w = jnp.exp(jnp.where(mask, s, -1e9))
                denom = jnp.sum(w, axis=1, keepdims=True)
                ctx = jnp.dot(w.astype(jnp.bfloat16), v_ref[:, sl],
                              preferred_element_type=jnp.float32)
                ctx_ref[:, sl] = (ctx / denom).astype(jnp.bfloat16)
            acc_ref[dst_slot, :, :] = jnp.dot(
                ctx_ref[:, :], wo_ref[:, :],
                preferred_element_type=jnp.float32).astype(jnp.bfloat16)

        compute_partial(0, 0)
        for s in range(N_DEV - 1):
            rdma = pltpu.make_async_remote_copy(
                src_ref=acc_ref.at[s],
                dst_ref=recv_ref.at[s],
                send_sem=rs_send.at[s],
                recv_sem=rs_recv.at[s],
                device_id=(right,),
                device_id_type=pl.DeviceIdType.MESH,
            )
            rdma.start()
            compute_partial(s + 1, s + 1)
            rdma.wait()
            acc_ref[s + 1, :, :] = (
                acc_ref[s + 1, :, :].astype(jnp.float32)
                + recv_ref[s, :, :].astype(jnp.float32)
            ).astype(jnp.bfloat16)

        own_off = lax.rem(my + 1, N_DEV) * CHUNK
        out_ref[pl.ds(own_off, CHUNK), :] = (
            acc_ref[N_DEV - 1, :, :].astype(jnp.float32))

        for t in range(N_DEV - 1):
            src_a = acc_ref.at[N_DEV - 1, :, pl.ds(0, HALF)] if t == 0 \
                else aga_ref.at[t - 1]
            rdma_a = pltpu.make_async_remote_copy(
                src_ref=src_a,
                dst_ref=aga_ref.at[t],
                send_sem=aga_send.at[t],
                recv_sem=aga_recv.at[t],
                device_id=(right,),
                device_id_type=pl.DeviceIdType.MESH,
            )
            src_b = acc_ref.at[N_DEV - 1, :, pl.ds(HALF, HALF)] if t == 0 \
                else agb_ref.at[t - 1]
            rdma_b = pltpu.make_async_remote_copy(
                src_ref=src_b,
                dst_ref=agb_ref.at[t],
                send_sem=agb_send.at[t],
                recv_sem=agb_recv.at[t],
                device_id=(left,),
                device_id_type=pl.DeviceIdType.MESH,
            )
            rdma_a.start()
            rdma_b.start()
            rdma_a.wait()
            rdma_b.wait()
            off_a = lax.rem(my - t + N_DEV, N_DEV) * CHUNK
            out_ref[pl.ds(off_a, CHUNK), 0:HALF] = (
                aga_ref[t, :, :].astype(jnp.float32))
            off_b = lax.rem(my + 2 + t, N_DEV) * CHUNK
            out_ref[pl.ds(off_b, CHUNK), HALF:D_OUT] = (
                agb_ref[t, :, :].astype(jnp.float32))

    out2 = pl.pallas_call(
        body,
        out_shape=jax.ShapeDtypeStruct((SQ, D_OUT), jnp.float32),
        in_specs=[pl.BlockSpec(memory_space=pltpu.VMEM)] * 5,
        out_specs=pl.BlockSpec(memory_space=pltpu.VMEM),
        scratch_shapes=[
            pltpu.VMEM((N_DEV, CHUNK, D_OUT), jnp.bfloat16),
            pltpu.VMEM((N_DEV - 1, CHUNK, D_OUT), jnp.bfloat16),
            pltpu.VMEM((N_DEV - 1, CHUNK, HALF), jnp.bfloat16),
            pltpu.VMEM((N_DEV - 1, CHUNK, HALF), jnp.bfloat16),
            pltpu.VMEM((CHUNK, D_LOCAL), jnp.bfloat16),
            pltpu.SemaphoreType.DMA((N_DEV - 1,)),
            pltpu.SemaphoreType.DMA((N_DEV - 1,)),
            pltpu.SemaphoreType.DMA((N_DEV - 1,)),
            pltpu.SemaphoreType.DMA((N_DEV - 1,)),
            pltpu.SemaphoreType.DMA((N_DEV - 1,)),
            pltpu.SemaphoreType.DMA((N_DEV - 1,)),
        ],
        compiler_params=pltpu.CompilerParams(
            collective_id=0,
            vmem_limit_bytes=100 * 1024 * 1024,
        ),
    )(x2, wq_s, k2, v2, wo_s)
    return out2.reshape(1, SQ, D_OUT)


# device time: 61230 ns/iter; 3.2351x vs baseline; 1.1454x over previous
import jax
import jax.numpy as jnp
from jax import lax
from jax.experimental import pallas as pl
from jax.experimental.pallas import tpu as pltpu

N_DEV = 4
SQ = 1024
SKV = 1024
H_PER = 8
DH = 128
D_LOCAL = H_PER * DH
D_OUT = 1024
SCALE = 0.08838834764831843

CHUNK = SQ // N_DEV
HALF = D_OUT // 2


def kernel(x, Wq, K_ext, V_ext, Wo):
    x2 = x.reshape(SQ, x.shape[-1])
    k2 = K_ext.reshape(SKV, D_LOCAL).astype(jnp.bfloat16)
    v2 = V_ext.reshape(SKV, D_LOCAL).astype(jnp.bfloat16)

    def body(x_ref, wq_ref, k_ref, v_ref, wo_ref, out_ref,
             acc_ref, recv_ref, nbl_ref, nbr_ref, dga_ref, dgb_ref, ctx_ref,
             stage_ref, wqf_ref, wof_ref, wqb_ref, wob_ref,
             rs_send, rs_recv, ag_send, ag_recv, cp_sem, w_sem):
        my = lax.axis_index("i")
        left = lax.rem(my + N_DEV - 1, N_DEV)
        right = lax.rem(my + 1, N_DEV)

        dma_wq = pltpu.make_async_copy(
            wq_ref.at[:, pl.ds(my * D_LOCAL, D_LOCAL)], wqf_ref, w_sem.at[0])
        dma_wo = pltpu.make_async_copy(
            wo_ref.at[pl.ds(my * D_LOCAL, D_LOCAL), :], wof_ref, w_sem.at[1])
        dma_wq.start()
        dma_wo.start()

        barrier_sem = pltpu.get_barrier_semaphore()
        for nbr in (left, right):
            pl.semaphore_signal(
                barrier_sem, inc=1,
                device_id=(nbr,), device_id_type=pl.DeviceIdType.MESH,
            )
        pl.semaphore_wait(barrier_sem, 2)

        dma_wq.wait()
        wqb_ref[:, :] = wqf_ref[:, :].astype(jnp.bfloat16)

        j_idx = lax.broadcasted_iota(jnp.int32, (CHUNK, SKV), 1)
        kb = j_idx // 64
        i_base = lax.broadcasted_iota(jnp.int32, (CHUNK, SKV), 0)

        def compute_partial(k_rel, dst_slot):
            off = (lax.rem(my - k_rel + N_DEV, N_DEV)) * CHUNK
            xc = x_ref[pl.ds(off, CHUNK), :].astype(jnp.bfloat16)
            qc = (jnp.dot(xc, wqb_ref[:, :], preferred_element_type=jnp.float32)
                  * SCALE).astype(jnp.bfloat16)
            qb = (i_base + off) // 64
            mask = (qb == kb) | (kb == 0) | (((qb + kb) % 3) == 0)
            for h in range(H_PER):
                sl = slice(h * DH, (h + 1) * DH)
                s = lax.dot_general(
                    qc[:, sl], k_ref[:, sl], (((1,), (1,)), ((), ())),
                    preferred_element_type=jnp.float32,
                )
                w = jnp.exp(jnp.where(mask, s, -1e9))
                denom = jnp.sum(w, axis=1, keepdims=True)
                ctx = jnp.dot(w.astype(jnp.bfloat16), v_ref[:, sl],
                              preferred_element_type=jnp.float32)
                ctx_ref[:, sl] = (ctx / denom).astype(jnp.bfloat16)
            if k_rel == 0:
                dma_wo.wait()
                wob_ref[:, :] = wof_ref[:, :].astype(jnp.bfloat16)
            acc_ref[dst_slot, :, :] = jnp.dot(
                ctx_ref[:, :], wob_ref[:, :],
                preferred_element_type=jnp.float32).astype(jnp.bfloat16)

        compute_partial(0, 0)
        for s in range(N_DEV - 1):
            rdma = pltpu.make_async_remote_copy(
                src_ref=acc_ref.at[s],
                dst_ref=recv_ref.at[s],
                send_sem=rs_send.at[s],
                recv_sem=rs_recv.at[s],
                device_id=(right,),
                device_id_type=pl.DeviceIdType.MESH,
            )
            rdma.start()
            compute_partial(s + 1, s + 1)
            rdma.wait()
            acc_ref[s + 1, :, :] = (
                acc_ref[s + 1, :, :].astype(jnp.float32)
                + recv_ref[s, :, :].astype(jnp.float32)
            ).astype(jnp.bfloat16)

        step1 = []
        for dev, dst in ((right, nbl_ref), (left, nbr_ref)):
            idx = 0 if dst is nbl_ref else 1
            r = pltpu.make_async_remote_copy(
                src_ref=acc_ref.at[N_DEV - 1],
                dst_ref=dst,
                send_sem=ag_send.at[idx],
                recv_sem=ag_recv.at[idx],
                device_id=(dev,),
                device_id_type=pl.DeviceIdType.MESH,
            )
            r.start()
            step1.append(r)
        own_off = lax.rem(my + 1, N_DEV) * CHUNK
        stage_ref[pl.ds(own_off, CHUNK), :] = (
            acc_ref[N_DEV - 1, :, :].astype(jnp.float32))
        for r in step1:
            r.wait()
        r_a = pltpu.make_async_remote_copy(
            src_ref=nbr_ref.at[:, pl.ds(0, HALF)],
            dst_ref=dga_ref,
            send_sem=ag_send.at[2],
            recv_sem=ag_recv.at[2],
            device_id=(left,),
            device_id_type=pl.DeviceIdType.MESH,
        )
        r_b = pltpu.make_async_remote_copy(
            src_ref=nbl_ref.at[:, pl.ds(HALF, HALF)],
            dst_ref=dgb_ref,
            send_sem=ag_send.at[3],
            recv_sem=ag_recv.at[3],
            device_id=(right,),
            device_id_type=pl.DeviceIdType.MESH,
        )
        r_a.start()
        r_b.start()
        off_l = my * CHUNK
        stage_ref[pl.ds(off_l, CHUNK), :] = nbl_ref[:, :].astype(jnp.float32)
        off_r = lax.rem(my + 2, N_DEV) * CHUNK
        stage_ref[pl.ds(off_r, CHUNK), :] = nbr_ref[:, :].astype(jnp.float32)
        r_a.wait()
        r_b.wait()
        off_d = lax.rem(my + 3, N_DEV) * CHUNK
        stage_ref[pl.ds(off_d, CHUNK), 0:HALF] = dga_ref[:, :].astype(jnp.float32)
        stage_ref[pl.ds(off_d, CHUNK), HALF:D_OUT] = (
            dgb_ref[:, :].astype(jnp.float32))
        cp = pltpu.make_async_copy(stage_ref, out_ref, cp_sem)
        cp.start()
        cp.wait()

    out2 = pl.pallas_call(
        body,
        out_shape=jax.ShapeDtypeStruct((SQ, D_OUT), jnp.float32),
        in_specs=[
            pl.BlockSpec(memory_space=pltpu.VMEM),
            pl.BlockSpec(memory_space=pl.ANY),
            pl.BlockSpec(memory_space=pltpu.VMEM),
            pl.BlockSpec(memory_space=pltpu.VMEM),
            pl.BlockSpec(memory_space=pl.ANY),
        ],
        out_specs=pl.BlockSpec(memory_space=pl.ANY),
        scratch_shapes=[
            pltpu.VMEM((N_DEV, CHUNK, D_OUT), jnp.bfloat16),
            pltpu.VMEM((N_DEV - 1, CHUNK, D_OUT), jnp.bfloat16),
            pltpu.VMEM((CHUNK, D_OUT), jnp.bfloat16),
            pltpu.VMEM((CHUNK, D_OUT), jnp.bfloat16),
            pltpu.VMEM((CHUNK, HALF), jnp.bfloat16),
            pltpu.VMEM((CHUNK, HALF), jnp.bfloat16),
            pltpu.VMEM((CHUNK, D_LOCAL), jnp.bfloat16),
            pltpu.VMEM((SQ, D_OUT), jnp.float32),
            pltpu.VMEM((SQ, D_LOCAL), jnp.float32),
            pltpu.VMEM((D_LOCAL, D_OUT), jnp.float32),
            pltpu.VMEM((SQ, D_LOCAL), jnp.bfloat16),
            pltpu.VMEM((D_LOCAL, D_OUT), jnp.bfloat16),
            pltpu.SemaphoreType.DMA((N_DEV - 1,)),
            pltpu.SemaphoreType.DMA((N_DEV - 1,)),
            pltpu.SemaphoreType.DMA((4,)),
            pltpu.SemaphoreType.DMA((4,)),
            pltpu.SemaphoreType.DMA,
            pltpu.SemaphoreType.DMA((2,)),
        ],
        compiler_params=pltpu.CompilerParams(
            collective_id=0,
            vmem_limit_bytes=100 * 1024 * 1024,
        ),
    )(x2, Wq, k2, v2, Wo)
    return out2.reshape(1, SQ, D_OUT)


# device time: 59305 ns/iter; 3.3401x vs baseline; 1.0325x over previous
import jax
import jax.numpy as jnp
from jax import lax
from jax.experimental import pallas as pl
from jax.experimental.pallas import tpu as pltpu

N_DEV = 4
SQ = 1024
SKV = 1024
H_PER = 8
DH = 128
D_LOCAL = H_PER * DH
D_OUT = 1024
SCALE = 0.08838834764831843

CHUNK = SQ // N_DEV
HALF = D_OUT // 2
COLA = pl.ds(0, HALF)
COLB = pl.ds(HALF, HALF)


def kernel(x, Wq, K_ext, V_ext, Wo):
    x2 = x.reshape(SQ, x.shape[-1])
    k2 = K_ext.reshape(SKV, D_LOCAL).astype(jnp.bfloat16)
    v2 = V_ext.reshape(SKV, D_LOCAL).astype(jnp.bfloat16)

    def body(x_ref, wq_ref, k_ref, v_ref, wo_ref, out_ref,
             acc_ref, rva_ref, rvb_ref,
             agla_ref, aglb_ref, agra_ref, agrb_ref, dg2a_ref, dg2b_ref,
             ctx_ref, stage_ref, wqf_ref, wof_ref, wqb_ref, wob_ref,
             rsa_send, rsa_recv, rsb_send, rsb_recv,
             ag_send, ag_recv, cp_sem, w_sem):
        my = lax.axis_index("i")
        left = lax.rem(my + N_DEV - 1, N_DEV)
        right = lax.rem(my + 1, N_DEV)

        dma_wq = pltpu.make_async_copy(
            wq_ref.at[:, pl.ds(my * D_LOCAL, D_LOCAL)], wqf_ref, w_sem.at[0])
        dma_wo = pltpu.make_async_copy(
            wo_ref.at[pl.ds(my * D_LOCAL, D_LOCAL), :], wof_ref, w_sem.at[1])
        dma_wq.start()
        dma_wo.start()

        barrier_sem = pltpu.get_barrier_semaphore()
        for nbr in (left, right):
            pl.semaphore_signal(
                barrier_sem, inc=1,
                device_id=(nbr,), device_id_type=pl.DeviceIdType.MESH,
            )
        pl.semaphore_wait(barrier_sem, 2)

        dma_wq.wait()
        wqb_ref[:, :] = wqf_ref[:, :].astype(jnp.bfloat16)

        j_idx = lax.broadcasted_iota(jnp.int32, (CHUNK, SKV), 1)
        kb = j_idx // 64
        i_base = lax.broadcasted_iota(jnp.int32, (CHUNK, SKV), 0)

        def compute_partial(chunk_delta, dst_slot):
            off = lax.rem(my + chunk_delta + N_DEV, N_DEV) * CHUNK
            xc = x_ref[pl.ds(off, CHUNK), :].astype(jnp.bfloat16)
            qc = (jnp.dot(xc, wqb_ref[:, :], preferred_element_type=jnp.float32)
                  * SCALE).astype(jnp.bfloat16)
            qb = (i_base + off) // 64
            mask = (qb == kb) | (kb == 0) | (((qb + kb) % 3) == 0)
            for h in range(H_PER):
                sl = slice(h * DH, (h + 1) * DH)
                s = lax.dot_general(
                    qc[:, sl], k_ref[:, sl], (((1,), (1,)), ((), ())),
                    preferred_element_type=jnp.float32,
                )
                w = jnp.exp(jnp.where(mask, s, -1e9))
                denom = jnp.sum(w, axis=1, keepdims=True)
                ctx = jnp.dot(w.astype(jnp.bfloat16), v_ref[:, sl],
                              preferred_element_type=jnp.float32)
                ctx_ref[:, sl] = (ctx / denom).astype(jnp.bfloat16)
            if dst_slot == 0:
                dma_wo.wait()
                wob_ref[:, :] = wof_ref[:, :].astype(jnp.bfloat16)
            acc_ref[dst_slot, :, :] = jnp.dot(
                ctx_ref[:, :], wob_ref[:, :],
                preferred_element_type=jnp.float32).astype(jnp.bfloat16)

        def rs_send(cols, src_slot, dst_ref, sem_s, sem_r, step, dev):
            r = pltpu.make_async_remote_copy(
                src_ref=acc_ref.at[src_slot, :, cols],
                dst_ref=dst_ref.at[step],
                send_sem=sem_s.at[step],
                recv_sem=sem_r.at[step],
                device_id=(dev,),
                device_id_type=pl.DeviceIdType.MESH,
            )
            r.start()
            return r

        def add_half(slot, cols, rv_ref, step):
            acc_ref[slot, :, cols] = (
                acc_ref[slot, :, cols].astype(jnp.float32)
                + rv_ref[step, :, :].astype(jnp.float32)
            ).astype(jnp.bfloat16)

        compute_partial(0, 0)
        cw0 = rs_send(COLA, 0, rva_ref, rsa_send, rsa_recv, 0, right)
        ccw0 = rs_send(COLB, 0, rvb_ref, rsb_send, rsb_recv, 0, left)
        compute_partial(-1, 1)
        cw0.wait()
        add_half(1, COLA, rva_ref, 0)
        cw1 = rs_send(COLA, 1, rva_ref, rsa_send, rsa_recv, 1, right)
        compute_partial(1, 2)
        ccw0.wait()
        add_half(2, COLB, rvb_ref, 0)
        ccw1 = rs_send(COLB, 2, rvb_ref, rsb_send, rsb_recv, 1, left)
        compute_partial(2, 3)
        cw1.wait()
        add_half(3, COLA, rva_ref, 1)
        cw2 = rs_send(COLA, 3, rva_ref, rsa_send, rsa_recv, 2, right)
        ccw1.wait()
        add_half(3, COLB, rvb_ref, 1)
        ccw2 = rs_send(COLB, 3, rvb_ref, rsb_send, rsb_recv, 2, left)
        cw2.wait()
        add_half(2, COLA, rva_ref, 2)
        ccw2.wait()
        add_half(1, COLB, rvb_ref, 2)

        def ag(src, dst, idx, dev):
            r = pltpu.make_async_remote_copy(
                src_ref=src, dst_ref=dst,
                send_sem=ag_send.at[idx], recv_sem=ag_recv.at[idx],
                device_id=(dev,), device_id_type=pl.DeviceIdType.MESH,
            )
            r.start()
            return r

        own_a = acc_ref.at[2, :, COLA]
        own_b = acc_ref.at[1, :, COLB]
        s1 = [
            ag(own_a, agla_ref, 0, right),
            ag(own_b, aglb_ref, 1, right),
            ag(own_a, agra_ref, 2, left),
            ag(own_b, agrb_ref, 3, left),
        ]
        off_oa = lax.rem(my + 1, N_DEV) * CHUNK
        stage_ref[pl.ds(off_oa, CHUNK), COLA] = own_a[:, :].astype(jnp.float32)
        off_ob = lax.rem(my + 3, N_DEV) * CHUNK
        stage_ref[pl.ds(off_ob, CHUNK), COLB] = own_b[:, :].astype(jnp.float32)
        for r in s1:
            r.wait()
        s2 = [
            ag(agla_ref, dg2a_ref, 4, right),
            ag(agrb_ref, dg2b_ref, 5, left),
        ]
        off_l = my * CHUNK
        stage_ref[pl.ds(off_l, CHUNK), COLA] = agla_ref[:, :].astype(jnp.float32)
        stage_ref[pl.ds(off_l, CHUNK), COLB] = agrb_ref[:, :].astype(jnp.float32)
        off_r = lax.rem(my + 2, N_DEV) * CHUNK
        stage_ref[pl.ds(off_r, CHUNK), COLA] = agra_ref[:, :].astype(jnp.float32)
        stage_ref[pl.ds(off_r, CHUNK), COLB] = aglb_ref[:, :].astype(jnp.float32)
        for r in s2:
            r.wait()
        off_d = lax.rem(my + 3, N_DEV) * CHUNK
        stage_ref[pl.ds(off_d, CHUNK), COLA] = dg2a_ref[:, :].astype(jnp.float32)
        off_d1 = lax.rem(my + 1, N_DEV) * CHUNK
        stage_ref[pl.ds(off_d1, CHUNK), COLB] = dg2b_ref[:, :].astype(jnp.float32)

        cp = pltpu.make_async_copy(stage_ref, out_ref, cp_sem)
        cp.start()
        cp.wait()

    out2 = pl.pallas_call(
        body,
        out_shape=jax.ShapeDtypeStruct((SQ, D_OUT), jnp.float32),
        in_specs=[
            pl.BlockSpec(memory_space=pltpu.VMEM),
            pl.BlockSpec(memory_space=pl.ANY),
            pl.BlockSpec(memory_space=pltpu.VMEM),
            pl.BlockSpec(memory_space=pltpu.VMEM),
            pl.BlockSpec(memory_space=pl.ANY),
        ],
        out_specs=pl.BlockSpec(memory_space=pl.ANY),
        scratch_shapes=[
            pltpu.VMEM((N_DEV, CHUNK, D_OUT), jnp.bfloat16),
            pltpu.VMEM((N_DEV - 1, CHUNK, HALF), jnp.bfloat16),
            pltpu.VMEM((N_DEV - 1, CHUNK, HALF), jnp.bfloat16),
            pltpu.VMEM((CHUNK, HALF), jnp.bfloat16),
            pltpu.VMEM((CHUNK, HALF), jnp.bfloat16),
            pltpu.VMEM((CHUNK, HALF), jnp.bfloat16),
            pltpu.VMEM((CHUNK, HALF), jnp.bfloat16),
            pltpu.VMEM((CHUNK, HALF), jnp.bfloat16),
            pltpu.VMEM((CHUNK, HALF), jnp.bfloat16),
            pltpu.VMEM((CHUNK, D_LOCAL), jnp.bfloat16),
            pltpu.VMEM((SQ, D_OUT), jnp.float32),
            pltpu.VMEM((SQ, D_LOCAL), jnp.float32),
            pltpu.VMEM((D_LOCAL, D_OUT), jnp.float32),
            pltpu.VMEM((SQ, D_LOCAL), jnp.bfloat16),
            pltpu.VMEM((D_LOCAL, D_OUT), jnp.bfloat16),
            pltpu.SemaphoreType.DMA((N_DEV - 1,)),
            pltpu.SemaphoreType.DMA((N_DEV - 1,)),
            pltpu.SemaphoreType.DMA((N_DEV - 1,)),
            pltpu.SemaphoreType.DMA((N_DEV - 1,)),
            pltpu.SemaphoreType.DMA((6,)),
            pltpu.SemaphoreType.DMA((6,)),
            pltpu.SemaphoreType.DMA,
            pltpu.SemaphoreType.DMA((2,)),
        ],
        compiler_params=pltpu.CompilerParams(
            collective_id=0,
            vmem_limit_bytes=100 * 1024 * 1024,
        ),
    )(x2, Wq, k2, v2, Wo)
    return out2.reshape(1, SQ, D_OUT)
